# baseline (speedup 1.0000x reference)
import math

import jax
import jax.numpy as jnp
from jax.experimental import pallas as pl
from jax.experimental.pallas import tpu as pltpu

N = 524288
D = 16
K = 8
RANGE_MIN = -3.0
RANGE_MAX = 3.0
MIN_BIN_SIZE = 1e-4
MIN_SLOPE = 1e-4

BL = 4096


def _prep(p_ref, t_ref):
    p = p_ref[...]
    wu = p[:, 0:K]
    hu = p[:, K:2 * K]
    su = p[:, 2 * K:3 * K + 1]

    total = RANGE_MAX - RANGE_MIN
    widths = jax.nn.softmax(wu, axis=-1) * (total - K * MIN_BIN_SIZE) + MIN_BIN_SIZE
    heights = jax.nn.softmax(hu, axis=-1) * (total - K * MIN_BIN_SIZE) + MIN_BIN_SIZE
    offset = math.log(math.exp(1.0 - MIN_SLOPE) - 1.0)
    slopes = jax.nn.softplus(su + offset) + MIN_SLOPE

    xp = [jnp.full((D, 1), RANGE_MIN, jnp.float32)]
    yp = [jnp.full((D, 1), RANGE_MIN, jnp.float32)]
    for j in range(K):
        xp.append(xp[-1] + widths[:, j:j + 1])
        yp.append(yp[-1] + heights[:, j:j + 1])
    iw = 1.0 / (widths + 1e-8)

    out = jnp.concatenate(
        xp + yp[:K] + [slopes, iw, heights,
                       jnp.zeros((D, 48 - 42), jnp.float32)], axis=-1)
    t_ref[...] = out


CH = 256


def _rqs_block(t_ref, x_ref, y_ref, ld_ref):
    t = t_ref[...]
    xp = [t[:, j:j + 1] for j in range(0, 9)]
    yp = [t[:, 9 + j:10 + j] for j in range(K)]
    sl = [t[:, 17 + j:18 + j] for j in range(K + 1)]
    iw_tab = [t[:, 26 + j:27 + j] for j in range(K)]
    h_tab = [t[:, 34 + j:35 + j] for j in range(K)]

    def chunk(c, carry):
        idx = pl.ds(c * CH, CH)
        x = x_ref[:, idx]
        m = [x >= xp[j] for j in range(1, K)]

        def gather(tab):
            v = jnp.broadcast_to(tab[0], x.shape)
            for j in range(1, K):
                v = jnp.where(m[j - 1], jnp.broadcast_to(tab[j], x.shape), v)
            return v

        x_k = gather(xp[:K])
        y_k = gather(yp)
        s_k = gather(sl[:K])
        s_k1 = gather(sl[1:K + 1])
        iw = gather(iw_tab)
        h = gather(h_tab)
        s = h * iw

        xi = jnp.clip((x - x_k) * iw, 0.0, 1.0)
        omx = 1.0 - xi
        u = xi * omx
        t2 = xi * xi
        num = s * t2 + s_k * u
        den0 = s + (s_k1 + s_k - 2.0 * s) * u
        den = jnp.maximum(jnp.abs(den0), 1e-8) * jnp.sign(den0)
        r = 1.0 / den
        y_sp = y_k + h * (num * r)
        dnum = (s * s) * (s_k1 * t2 + 2.0 * s * u + s_k * (omx * omx))
        deriv = jnp.maximum(dnum * r * r, 1e-8)

        below = x < RANGE_MIN
        above = x > RANGE_MAX
        sl0 = jnp.broadcast_to(sl[0], x.shape)
        sl8 = jnp.broadcast_to(sl[K], x.shape)
        y_lin_l = (x - RANGE_MIN) * sl0 + RANGE_MIN
        y_lin_r = (x - RANGE_MAX) * sl8 + RANGE_MAX
        y_ref[:, idx] = jnp.where(below, y_lin_l, jnp.where(above, y_lin_r, y_sp))
        dsel = jnp.where(below, sl0, jnp.where(above, sl8, deriv))
        ld_ref[idx] = jnp.sum(jnp.log(dsel), axis=0)
        return carry

    jax.lax.fori_loop(0, BL // CH, chunk, 0)


def kernel(x, params):
    xt = x.T
    tables = pl.pallas_call(
        _prep,
        out_shape=jax.ShapeDtypeStruct((D, 48), jnp.float32),
    )(params)
    yt, ld = pl.pallas_call(
        _rqs_block,
        grid=(N // BL,),
        in_specs=[
            pl.BlockSpec((D, 48), lambda i: (0, 0)),
            pl.BlockSpec((D, BL), lambda i: (0, i)),
        ],
        out_specs=[
            pl.BlockSpec((D, BL), lambda i: (0, i)),
            pl.BlockSpec((BL,), lambda i: (i,)),
        ],
        out_shape=[
            jax.ShapeDtypeStruct((D, N), jnp.float32),
            jax.ShapeDtypeStruct((N,), jnp.float32),
        ],
        compiler_params=pltpu.CompilerParams(
            dimension_semantics=("parallel",),
        ),
    )(tables, xt)
    return yt.T, ld
